# Initial kernel scaffold; baseline (speedup 1.0000x reference)
import jax
import jax.numpy as jnp
from jax.experimental import pallas as pl


def kernel(x_embed, prompt_key, new_desc_embed, w_text, w_prompt_proj, w_qkv_k, w_proj_k, b_proj_k, w_qkv_v, w_proj_v, b_proj_v):
    raise NotImplementedError("write your pallas kernel here")


# speedup vs baseline: 5.3876x; 5.3876x over previous
import functools

import jax
import jax.numpy as jnp
from jax import lax
from jax.experimental import pallas as pl
from jax.experimental.pallas import tpu as pltpu

_B, _S, _D = 4, 2048, 768
_H, _HD = 12, 64
_TKL, _LEN = 3, 5
_LMAX = 100
_LPAD = 112
_NEG = -3e38
_BIGI = 2**30

_SCHUNK = 256
_GS = _S // _SCHUNK


def _tc_a_body(x_ref, pk_ref, nd_ref, wt_ref, sim_ref, de_ref, dout_ref, acc_ref):
    g = pl.program_id(0)

    @pl.when(g == 0)
    def _():
        acc_ref[...] = jnp.zeros_like(acc_ref)

    acc_ref[...] += jnp.sum(x_ref[...], axis=1)

    @pl.when(g == _GS - 1)
    def _():
        x_mean = acc_ref[...] * (1.0 / _S)
        ss = jnp.sum(x_mean * x_mean, axis=1, keepdims=True)
        x_norm = x_mean * lax.rsqrt(jnp.maximum(ss, 1e-12))
        pk = pk_ref[...]
        pss = jnp.sum(pk * pk, axis=1, keepdims=True)
        pk_norm = pk * lax.rsqrt(jnp.maximum(pss, 1e-12))
        sim = lax.dot_general(x_norm, pk_norm, (((1,), (1,)), ((), ())),
                              preferred_element_type=jnp.float32)
        col = lax.broadcasted_iota(jnp.int32, sim.shape, 1)
        sim = jnp.where(col < _LMAX, sim, _NEG)
        sim_ref[...] = sim
        de = lax.dot_general(nd_ref[...], wt_ref[...], (((1,), (1,)), ((), ())),
                             preferred_element_type=jnp.float32)
        de_ref[...] = de
        w = jnp.zeros_like(sim)
        s = sim
        for _ in range(_TKL):
            m = jnp.max(s, axis=1, keepdims=True)
            cand = jnp.where(s == m, col, _BIGI)
            a = jnp.min(cand, axis=1, keepdims=True)
            hit = (col == a)
            w = w + jnp.where(hit, m, 0.0)
            s = jnp.where(hit, _NEG, s)
        dout_ref[...] = lax.dot_general(w, de, (((1,), (0,)), ((), ())),
                                        preferred_element_type=jnp.float32)


def _tc_c_body(dout_ref, sim_ref, wpp_ref, wv_k_ref, wp_k_ref, bp_k_ref,
               wv_v_ref, wp_v_ref, bp_v_ref, out_ref):
    bp = lax.dot_general(dout_ref[...], wpp_ref[...], (((1,), (1,)), ((), ())),
                         preferred_element_type=jnp.float32)
    sim = sim_ref[...]
    col = lax.broadcasted_iota(jnp.int32, sim.shape, 1)
    s3 = [jnp.sum(jnp.where(col == p, sim, 0.0), axis=1, keepdims=True)
          for p in range(_TKL)]
    acc_k_list = []
    acc_v_list = []
    for h in range(_H):
        bh = bp[:, h * _HD:(h + 1) * _HD]
        acc_k = jnp.zeros((_B, _HD), jnp.float32)
        acc_v = jnp.zeros((_B, _HD), jnp.float32)
        for p in range(_TKL):
            vk = lax.dot_general(bh, wv_k_ref[h, p], (((1,), (1,)), ((), ())),
                                 preferred_element_type=jnp.float32)
            ok = lax.dot_general(vk, wp_k_ref[h, p], (((1,), (1,)), ((), ())),
                                 preferred_element_type=jnp.float32)
            ok = ok + bp_k_ref[h, p][None, :]
            vv = lax.dot_general(bh, wv_v_ref[h, p], (((1,), (1,)), ((), ())),
                                 preferred_element_type=jnp.float32)
            ov = lax.dot_general(vv, wp_v_ref[h, p], (((1,), (1,)), ((), ())),
                                 preferred_element_type=jnp.float32)
            ov = ov + bp_v_ref[h, p][None, :]
            acc_k = acc_k + s3[p] * ok
            acc_v = acc_v + s3[p] * ov
        acc_k_list.append(acc_k)
        acc_v_list.append(acc_v)
    nk_flat = jnp.concatenate(acc_k_list, axis=1)
    nv_flat = jnp.concatenate(acc_v_list, axis=1)
    out = jnp.concatenate(
        [jnp.broadcast_to(nk_flat[:, None, :], (_B, _LEN, _D)),
         jnp.broadcast_to(nv_flat[:, None, :], (_B, _LEN, _D))], axis=1)
    out_ref[...] = out


def kernel(x_embed, prompt_key, new_desc_embed, w_text, w_prompt_proj,
           w_qkv_k, w_proj_k, b_proj_k, w_qkv_v, w_proj_v, b_proj_v):
    nd_pad = jnp.pad(new_desc_embed, ((0, _LPAD - _LMAX), (0, 0)))

    sim_p, de, desc_out = pl.pallas_call(
        _tc_a_body,
        grid=(_GS,),
        in_specs=[
            pl.BlockSpec((_B, _SCHUNK, _D), lambda g: (0, g, 0)),
            pl.BlockSpec((_LPAD, _D), lambda g: (0, 0)),
            pl.BlockSpec((_LPAD, _D), lambda g: (0, 0)),
            pl.BlockSpec((_D, _D), lambda g: (0, 0)),
        ],
        out_specs=[
            pl.BlockSpec((_B, _LPAD), lambda g: (0, 0)),
            pl.BlockSpec((_LPAD, _D), lambda g: (0, 0)),
            pl.BlockSpec((_B, _D), lambda g: (0, 0)),
        ],
        out_shape=[
            jax.ShapeDtypeStruct((_B, _LPAD), jnp.float32),
            jax.ShapeDtypeStruct((_LPAD, _D), jnp.float32),
            jax.ShapeDtypeStruct((_B, _D), jnp.float32),
        ],
        scratch_shapes=[pltpu.VMEM((_B, _D), jnp.float32)],
    )(x_embed, prompt_key, nd_pad, w_text)

    del de

    out_bp = pl.pallas_call(
        _tc_c_body,
        grid=(1,),
        in_specs=[
            pl.BlockSpec((_B, _D), lambda i: (0, 0)),
            pl.BlockSpec((_B, _LPAD), lambda i: (0, 0)),
            pl.BlockSpec((_D, _D), lambda i: (0, 0)),
            pl.BlockSpec((_H, _TKL, _HD, _HD), lambda i: (0, 0, 2, 0)),
            pl.BlockSpec((_H, _TKL, _HD, _HD), lambda i: (0, 0, 0, 0)),
            pl.BlockSpec((_H, _TKL, _HD), lambda i: (0, 0, 0)),
            pl.BlockSpec((_H, _TKL, _HD, _HD), lambda i: (0, 0, 2, 0)),
            pl.BlockSpec((_H, _TKL, _HD, _HD), lambda i: (0, 0, 0, 0)),
            pl.BlockSpec((_H, _TKL, _HD), lambda i: (0, 0, 0)),
        ],
        out_specs=pl.BlockSpec((_B, 2 * _LEN, _D), lambda i: (0, 0, 0)),
        out_shape=jax.ShapeDtypeStruct((_B, 2 * _LEN, _D), jnp.float32),
    )(desc_out, sim_p, w_prompt_proj, w_qkv_k, w_proj_k, b_proj_k,
      w_qkv_v, w_proj_v, b_proj_v)

    similarity = sim_p[:, :_LMAX]
    return (similarity, desc_out, out_bp)
